# baseline (speedup 1.0000x reference)
import functools

import jax
import jax.numpy as jnp
from jax import lax
from jax.experimental import pallas as pl
from jax.experimental.pallas import tpu as pltpu
from jax.experimental.pallas import tpu_sc as plsc

NNODE = 50000
NPAD = 51200
E0 = 800000
EPAD = 802816
NC, NS = 2, 16
CHUNK = 128
DHALF = 32
ROWS_PT = NPAD // NS
EDGES_PT_CONV = EPAD // NS
NCHUNK_CONV = EDGES_PT_CONV // CHUNK
SUPER = 2
SEDGE = SUPER * CHUNK
NSUP = EDGES_PT_CONV // SEDGE
NBUF = 3
IBUF = 4
IDXR = EPAD // CHUNK
ROWS_PT_IDX = EDGES_PT_CONV // CHUNK
EDGES_PT_P0 = EPAD // (NC * NS)
NCHUNK_P0 = EDGES_PT_P0 // CHUNK
SUP0 = 7
NSUP0 = NCHUNK_P0 // SUP0
ZROWS = 50
WCH = 128
RBLK = 2048
GRID_N = NPAD // RBLK
QPAD = NPAD // 4
QBLK = RBLK // 4

def _sc_mesh():
    return plsc.VectorSubcoreMesh(
        core_axis_name="c", subcore_axis_name="s",
        num_cores=NC, num_subcores=NS)


def _leaky(v):
    return jnp.where(v >= 0, v, 0.01 * v)


def _slot(n):
    return (n & -2048) | ((n & 511) << 2) | ((n >> 9) & 3)


def _phase0(src2, dst2):
    @functools.partial(
        pl.kernel,
        out_type=(
            jax.ShapeDtypeStruct((IDXR, CHUNK), jnp.int32),
            jax.ShapeDtypeStruct((2 * IDXR, CHUNK), jnp.int32),
            jax.ShapeDtypeStruct((NC * NPAD,), jnp.float32),
        ),
        mesh=_sc_mesh(),
        compiler_params=pltpu.CompilerParams(use_tc_tiling_on_sc=False),
        scratch_types=[
            pltpu.VMEM((SUP0, CHUNK), jnp.int32),
            pltpu.VMEM((SUP0, CHUNK), jnp.int32),
            pltpu.VMEM((2, SUP0, CHUNK), jnp.int32),
            pltpu.VMEM((2, SUP0, CHUNK), jnp.int32),
            pltpu.VMEM((2, SUP0, CHUNK), jnp.int32),
            pltpu.VMEM((2, SUP0, CHUNK), jnp.int32),
            pltpu.VMEM((CHUNK,), jnp.float32),
            pltpu.VMEM((ROWS_PT,), jnp.float32),
            pltpu.VMEM_SHARED((NPAD,), jnp.float32),
            pltpu.SemaphoreType.DMA,
            pltpu.SemaphoreType.DMA,
            pltpu.SemaphoreType.DMA,
        ],
    )
    def k(src_hbm, dst_hbm, dstm2, srcb2, deg_hbm,
          src_v, dst_v, soff_v, shi_v, srcm_v, dstm_v, ones_v, zbuf_v, deg_sh,
          lsem, stsem, scsem):
        c = lax.axis_index("c")
        s = lax.axis_index("s")
        lane = lax.iota(jnp.int32, 16)
        dummy = NNODE + (lane & 7)

        for j in range(CHUNK // 16):
            ones_v[pl.ds(j * 16, 16)] = jnp.full((16,), 1.0, jnp.float32)

        def zfill(i, _):
            zbuf_v[pl.ds(i * 16, 16)] = jnp.zeros((16,), jnp.float32)
            return 0
        lax.fori_loop(0, ROWS_PT // 16, zfill, 0)

        pltpu.sync_copy(zbuf_v, deg_sh.at[pl.ds(s * ROWS_PT, ROWS_PT)])
        plsc.subcore_barrier()

        base_row = (c * NS + s) * NCHUNK_P0

        def fire_stores(i, b):
            r0 = base_row + i * SUP0
            pltpu.async_copy(dstm_v.at[b], dstm2.at[pl.ds(r0, SUP0)], stsem)
            pltpu.async_copy(soff_v.at[b], srcb2.at[pl.ds(r0, SUP0)], stsem)
            pltpu.async_copy(shi_v.at[b],
                             srcb2.at[pl.ds(IDXR + r0, SUP0)], stsem)

        def drain_stores(i, b):
            r0 = base_row + i * SUP0
            pltpu.make_async_copy(
                dstm_v.at[b], dstm2.at[pl.ds(r0, SUP0)], stsem).wait()
            pltpu.make_async_copy(
                soff_v.at[b], srcb2.at[pl.ds(r0, SUP0)], stsem).wait()
            pltpu.make_async_copy(
                shi_v.at[b], srcb2.at[pl.ds(IDXR + r0, SUP0)], stsem).wait()

        def fire_deg(b):
            for kk in range(SUP0):
                pltpu.async_copy(ones_v, deg_sh.at[srcm_v.at[b, kk]], scsem,
                                 add=True)

        def drain_deg(b):
            for kk in range(SUP0):
                pltpu.make_async_copy(
                    ones_v, deg_sh.at[srcm_v.at[b, kk]], scsem).wait()

        def step(i, b):
            r0 = base_row + i * SUP0
            @pl.when(i >= 2)
            def _():
                drain_stores(i - 2, b)
                drain_deg(b)
            pltpu.async_copy(src_hbm.at[pl.ds(r0, SUP0)], src_v, lsem)
            pltpu.async_copy(dst_hbm.at[pl.ds(r0, SUP0)], dst_v, lsem)
            pltpu.make_async_copy(
                src_hbm.at[pl.ds(r0, SUP0)], src_v, lsem).wait()
            pltpu.make_async_copy(
                dst_hbm.at[pl.ds(r0, SUP0)], dst_v, lsem).wait()
            for kk in range(SUP0):
                for j in range(CHUNK // 16):
                    sl = pl.ds(j * 16, 16)
                    sv = src_v[kk, sl]
                    dv = dst_v[kk, sl]
                    m = sv == dv
                    ss = _slot(sv)
                    soff_v[b, kk, sl] = ss
                    shi_v[b, kk, sl] = ss + NPAD
                    srcm_v[b, kk, sl] = jnp.where(m, dummy, sv)
                    dstm_v[b, kk, sl] = _slot(jnp.where(m, dummy, dv))
            fire_stores(i, b)
            fire_deg(b)

        def outer(o, _):
            step(2 * o, 0)
            step(2 * o + 1, 1)
            return 0
        lax.fori_loop(0, NSUP0 // 2, outer, 0)

        drain_stores(NSUP0 - 2, 0)
        drain_deg(0)
        drain_stores(NSUP0 - 1, 1)
        drain_deg(1)

        plsc.subcore_barrier()
        pltpu.sync_copy(deg_sh.at[pl.ds(s * ROWS_PT, ROWS_PT)], zbuf_v)
        pltpu.sync_copy(zbuf_v,
                        deg_hbm.at[pl.ds(c * NPAD + s * ROWS_PT, ROWS_PT)])

    return k(src2, dst2)


def _conv_sc(y_flat, srcb2, dstm2):
    @functools.partial(
        pl.kernel,
        out_type=jax.ShapeDtypeStruct((NC * NPAD, DHALF), jnp.float32),
        mesh=_sc_mesh(),
        compiler_params=pltpu.CompilerParams(use_tc_tiling_on_sc=False),
        scratch_types=[
            pltpu.VMEM((IBUF, SUPER, CHUNK), jnp.int32),
            pltpu.VMEM((IBUF, SUPER, CHUNK), jnp.int32),
            pltpu.VMEM((NBUF, SEDGE, DHALF), jnp.float32),
            pltpu.VMEM((ZROWS, DHALF), jnp.float32),
            pltpu.VMEM_SHARED((NPAD, DHALF), jnp.float32),
            pltpu.SemaphoreType.DMA,
            pltpu.SemaphoreType.DMA,
            pltpu.SemaphoreType.DMA,
        ],
    )
    def k(y_hbm, src2, dst2, out_hbm,
          sidx_v, didx_v, rows_v, zbuf_v, acc_sh, isem, gsem, ssem):
        c = lax.axis_index("c")
        s = lax.axis_index("s")

        def zfill(i, _):
            zbuf_v[i, pl.ds(0, 16)] = jnp.zeros((16,), jnp.float32)
            zbuf_v[i, pl.ds(16, 16)] = jnp.zeros((16,), jnp.float32)
            return 0
        lax.fori_loop(0, ZROWS, zfill, 0)

        def zrow(t):
            return acc_sh.at[pl.ds(s * ROWS_PT + t * ZROWS, ZROWS)]

        def zfire(t, _):
            pltpu.async_copy(zbuf_v, zrow(t), ssem)
            return 0
        lax.fori_loop(0, ROWS_PT // ZROWS, zfire, 0)

        def zdrain(t, _):
            pltpu.make_async_copy(zbuf_v, zrow(t), ssem).wait()
            return 0
        lax.fori_loop(0, ROWS_PT // ZROWS, zdrain, 0)
        plsc.subcore_barrier()

        base_row = s * ROWS_PT_IDX

        def fire_idx(j, bi):
            r0 = base_row + j * SUPER
            pltpu.async_copy(src2.at[pl.ds(c * IDXR + r0, SUPER)],
                             sidx_v.at[bi], isem)
            pltpu.async_copy(dst2.at[pl.ds(r0, SUPER)], didx_v.at[bi], isem)

        def wait_idx(j, bi):
            r0 = base_row + j * SUPER
            pltpu.make_async_copy(src2.at[pl.ds(c * IDXR + r0, SUPER)],
                                  sidx_v.at[bi], isem).wait()
            pltpu.make_async_copy(dst2.at[pl.ds(r0, SUPER)], didx_v.at[bi],
                                  isem).wait()

        def fire_gathers(b, bi):
            for kk in range(SUPER):
                pltpu.async_copy(
                    y_hbm.at[sidx_v.at[bi, kk]],
                    rows_v.at[b, pl.ds(kk * CHUNK, CHUNK)], gsem)

        def drain_gathers(b, bi):
            pltpu.make_async_copy(
                y_hbm.at[pl.ds(0, SEDGE)], rows_v.at[b], gsem).wait()

        def fire_scatters(b, bi):
            for kk in range(SUPER):
                pltpu.async_copy(
                    rows_v.at[b, pl.ds(kk * CHUNK, CHUNK)],
                    acc_sh.at[didx_v.at[bi, kk]], ssem, add=True)

        def drain_scatters(b, bi):
            pltpu.make_async_copy(
                rows_v.at[b], acc_sh.at[pl.ds(0, SEDGE)], ssem).wait()

        def step(i, p):
            b = p % NBUF
            b2 = (p + NBUF - 2) % NBUF
            bi = p % IBUF
            bi2 = (p + IBUF - 2) % IBUF
            bi3 = (p + IBUF - 3) % IBUF
            bin1 = (p + 1) % IBUF
            @pl.when(jnp.logical_and(i >= 3, i <= NSUP + 2))
            def _():
                drain_scatters(b, bi3)
            @pl.when(jnp.logical_and(i >= 2, i < NSUP + 2))
            def _():
                drain_gathers(b2, bi2)
                fire_scatters(b2, bi2)
            @pl.when(i + 1 < NSUP)
            def _():
                fire_idx(i + 1, bin1)
            @pl.when(i < NSUP)
            def _():
                wait_idx(i, bi)
                fire_gathers(b, bi)

        fire_idx(0, 0)

        def outer(o, _):
            for p in range(NBUF * IBUF):
                step(NBUF * IBUF * o + p, p)
            return 0
        lax.fori_loop(0, (NSUP + 2 + NBUF * IBUF) // (NBUF * IBUF), outer, 0)
        plsc.subcore_barrier()

        NW = ROWS_PT // WCH
        def wslot(t):
            return rows_v.at[t % NBUF,
                             pl.ds(((t // NBUF) % SUPER) * CHUNK, CHUNK)]
        for t in range(NW):
            r0 = s * ROWS_PT + t * WCH
            ho = c * NPAD + r0
            if t >= NBUF * SUPER:
                po = c * NPAD + s * ROWS_PT + (t - NBUF * SUPER) * WCH
                pltpu.make_async_copy(
                    wslot(t - NBUF * SUPER),
                    out_hbm.at[pl.ds(po, WCH)], isem).wait()
            pltpu.sync_copy(acc_sh.at[pl.ds(r0, WCH)], wslot(t))
            pltpu.async_copy(wslot(t), out_hbm.at[pl.ds(ho, WCH)], isem)
        for t in range(max(0, NW - NBUF * SUPER), NW):
            ho = c * NPAD + s * ROWS_PT + t * WCH
            pltpu.make_async_copy(
                wslot(t), out_hbm.at[pl.ds(ho, WCH)], isem).wait()

    return k(y_flat, srcb2, dstm2)


def _mlp_tc(features, mlp_w, mlp_b2):
    def body(f_ref, w_ref, b_ref, o_ref):
        o_ref[...] = (
            jnp.dot(f_ref[...], w_ref[...], preferred_element_type=jnp.float32)
            + b_ref[...])
    return pl.pallas_call(
        body,
        grid=(20,),
        in_specs=[
            pl.BlockSpec((2000, 128), lambda i: (i, 0)),
            pl.BlockSpec((128, 128), lambda i: (0, 0)),
            pl.BlockSpec((1, 128), lambda i: (0, 0)),
        ],
        out_specs=pl.BlockSpec((2000, 128), lambda i: (i, 0)),
        out_shape=jax.ShapeDtypeStruct((40000, 128), jnp.float32),
    )(features, mlp_w, mlp_b2)


def _dinv_of(deg_ref):
    deg = deg_ref[0] + deg_ref[1] + 1.0
    return lax.rsqrt(deg)[:, None]


def _l2n(h):
    s = jnp.sum(h * h, axis=1, keepdims=True)
    return h * lax.rsqrt(jnp.maximum(s, 1e-24))


def _split_out(o_ref, y):
    q = y.shape[0] // 4
    for c in range(2):
        cols = y[:, c * DHALF:(c + 1) * DHALF]
        o_ref[c] = jnp.concatenate(
            [cols[j * q:(j + 1) * q, :] for j in range(4)], axis=1)


def _unpack(h):
    return jnp.concatenate(
        [h[:, j * DHALF:(j + 1) * DHALF] for j in range(4)], axis=0)


def _cat2(ref):
    return jnp.concatenate([_unpack(ref[0]), _unpack(ref[1])], axis=1)


def _cat2sum(a_ref, y_ref):
    return jnp.concatenate(
        [_unpack(a_ref[0] + y_ref[0]), _unpack(a_ref[1] + y_ref[1])], axis=1)


def _layer0_tc(xcat, deg_part, cw1):
    def body(x_ref, deg_ref, w_ref, o_ref):
        dinv = _dinv_of(deg_ref)
        xn = _l2n(x_ref[...])
        y = jnp.dot(xn, w_ref[...], preferred_element_type=jnp.float32) * dinv
        _split_out(o_ref, y)
    return pl.pallas_call(
        body,
        grid=(GRID_N,),
        in_specs=[
            pl.BlockSpec((RBLK, 128), lambda i: (i, 0)),
            pl.BlockSpec((2, RBLK), lambda i: (0, i)),
            pl.BlockSpec((128, 64), lambda i: (0, 0)),
        ],
        out_specs=pl.BlockSpec((2, QBLK, 128), lambda i: (0, i, 0)),
        out_shape=jax.ShapeDtypeStruct((2, QPAD, 128), jnp.float32),
    )(xcat, deg_part, cw1)


def _layer_tc(acc, y, deg_part, cb2, gw, gb2, cwn):
    def body(a_ref, y_ref, deg_ref, cb_ref, gw_ref, gb_ref, wn_ref, o_ref):
        dinv = _dinv_of(deg_ref)
        h = dinv * _cat2sum(a_ref, y_ref) + cb_ref[...]
        h = _leaky(_l2n(h))
        x = _leaky(
            jnp.dot(h, gw_ref[...], preferred_element_type=jnp.float32)
            + gb_ref[...])
        yn = jnp.dot(x, wn_ref[...], preferred_element_type=jnp.float32) * dinv
        _split_out(o_ref, yn)
    blk2 = pl.BlockSpec((2, QBLK, 128), lambda i: (0, i, 0))
    w64 = pl.BlockSpec((64, 64), lambda i: (0, 0))
    b64 = pl.BlockSpec((1, 64), lambda i: (0, 0))
    return pl.pallas_call(
        body,
        grid=(GRID_N,),
        in_specs=[blk2, blk2, pl.BlockSpec((2, RBLK), lambda i: (0, i)),
                  b64, w64, b64, w64],
        out_specs=blk2,
        out_shape=jax.ShapeDtypeStruct((2, QPAD, 128), jnp.float32),
    )(acc, y, deg_part, cb2, gw, gb2, cwn)


def _layer3_tc(acc, y, deg_part, cb2, gw, gb2, cw4, cw5):
    def body(a_ref, y_ref, deg_ref, cb_ref, gw_ref, gb_ref, w4_ref, w5_ref,
             o4_ref, o5_ref, ox_ref):
        dinv = _dinv_of(deg_ref)
        h = dinv * _cat2sum(a_ref, y_ref) + cb_ref[...]
        h = _leaky(_l2n(h))
        x = _leaky(
            jnp.dot(h, gw_ref[...], preferred_element_type=jnp.float32)
            + gb_ref[...])
        y4 = jnp.dot(x, w4_ref[...], preferred_element_type=jnp.float32) * dinv
        y5 = jnp.dot(x, w5_ref[...], preferred_element_type=jnp.float32) * dinv
        _split_out(o4_ref, y4)
        _split_out(o5_ref, y5)
        ox_ref[...] = x.T
    blk2 = pl.BlockSpec((2, QBLK, 128), lambda i: (0, i, 0))
    w64 = pl.BlockSpec((64, 64), lambda i: (0, 0))
    b64 = pl.BlockSpec((1, 64), lambda i: (0, 0))
    return pl.pallas_call(
        body,
        grid=(GRID_N,),
        in_specs=[blk2, blk2, pl.BlockSpec((2, RBLK), lambda i: (0, i)),
                  b64, w64, b64, w64, w64],
        out_specs=[blk2, blk2, pl.BlockSpec((64, RBLK), lambda i: (0, i))],
        out_shape=[
            jax.ShapeDtypeStruct((2, QPAD, 128), jnp.float32),
            jax.ShapeDtypeStruct((2, QPAD, 128), jnp.float32),
            jax.ShapeDtypeStruct((64, NPAD), jnp.float32),
        ],
    )(acc, y, deg_part, cb2, gw, gb2, cw4, cw5)


def _head_tc(acc, y, xt3, deg_part, cb, gw, gb, lw, lb):
    def body(a_ref, yh_ref, x_ref, deg_ref, cb_ref, gw_ref, gb_ref,
             lw_ref, lb_ref, o_ref):
        dinv = _dinv_of(deg_ref)
        h = dinv * _cat2sum(a_ref, yh_ref) + cb_ref[...]
        ht = _leaky(_l2n(h)).T
        xt = x_ref[...]
        dnt = (((0,), (0,)), ((), ()))
        xht = _leaky(
            lax.dot_general(lw_ref[...], xt, dnt,
                            preferred_element_type=jnp.float32) + lb_ref[...])
        o_ref[...] = (
            lax.dot_general(gw_ref[...], ht, dnt,
                            preferred_element_type=jnp.float32)
            + gb_ref[...] + xht)
    blk2 = pl.BlockSpec((2, QBLK, 128), lambda i: (0, i, 0))
    blkxt = pl.BlockSpec((64, RBLK), lambda i: (0, i))
    w64 = pl.BlockSpec((64, 64), lambda i: (0, 0))
    b64t = pl.BlockSpec((64, 1), lambda i: (0, 0))
    return pl.pallas_call(
        body,
        grid=(GRID_N,),
        in_specs=[blk2, blk2, blkxt,
                  pl.BlockSpec((2, RBLK), lambda i: (0, i)),
                  pl.BlockSpec((1, 64), lambda i: (0, 0)),
                  w64, b64t, w64, b64t],
        out_specs=pl.BlockSpec((64, RBLK), lambda i: (0, i)),
        out_shape=jax.ShapeDtypeStruct((64, NNODE), jnp.float32),
    )(acc, y, xt3, deg_part, cb, gw, gb, lw, lb).T


def kernel(features, edge_index, preference, mlp_w, mlp_b,
           cw1, cb1, cw2, cb2, cw3, cb3, cw4, cb4, cw5, cb5,
           lw1, lb1, lw2, lb2, lw3, lb3, lw4, lb4, lw5, lb5,
           gw1, gb1, gw2, gb2, gw3, gb3, gw4, gb4, gw5, gb5):
    pad = NNODE + (jnp.arange(EPAD - E0, dtype=jnp.int32) % 8)
    src_p = jnp.concatenate([edge_index[0], pad]).reshape(IDXR, CHUNK)
    dst_p = jnp.concatenate([edge_index[1], pad]).reshape(IDXR, CHUNK)

    dstm2, srcb2, deg_flat = _phase0(src_p, dst_p)
    deg_part = deg_flat.reshape(2, NPAD)

    r2 = lambda b: b.reshape(1, -1)

    temp = _mlp_tc(features, mlp_w, r2(mlp_b))
    xcat = jnp.concatenate(
        [preference, temp, jnp.zeros((NPAD - NNODE, 128), jnp.float32)], axis=0)

    def sc_in(y):
        return y.reshape(NC * NPAD, DHALF)

    def tc_in(a):
        return a.reshape(2, QPAD, 128)

    y1 = _layer0_tc(xcat, deg_part, cw1)
    acc1 = _conv_sc(sc_in(y1), srcb2, dstm2)
    y2 = _layer_tc(tc_in(acc1), y1, deg_part, r2(cb1), gw1, r2(gb1), cw2)
    acc2 = _conv_sc(sc_in(y2), srcb2, dstm2)
    y3 = _layer_tc(tc_in(acc2), y2, deg_part, r2(cb2), gw2, r2(gb2), cw3)
    acc3 = _conv_sc(sc_in(y3), srcb2, dstm2)
    y4, y5, x3 = _layer3_tc(tc_in(acc3), y3, deg_part,
                            r2(cb3), gw3, r2(gb3), cw4, cw5)
    acc4 = _conv_sc(sc_in(y4), srcb2, dstm2)
    acc5 = _conv_sc(sc_in(y5), srcb2, dstm2)
    c2 = lambda b: b.reshape(-1, 1)
    mu = _head_tc(tc_in(acc4), y4, x3, deg_part,
                  r2(cb4), gw4, c2(gb4), lw4, c2(lb4))
    lv = _head_tc(tc_in(acc5), y5, x3, deg_part,
                  r2(cb5), gw5, c2(gb5), lw5, c2(lb5))
    return (mu, lv)


# speedup vs baseline: 1.0018x; 1.0018x over previous
import functools

import jax
import jax.numpy as jnp
from jax import lax
from jax.experimental import pallas as pl
from jax.experimental.pallas import tpu as pltpu
from jax.experimental.pallas import tpu_sc as plsc

NNODE = 50000
NPAD = 51200
E0 = 800000
EPAD = 802816
NC, NS = 2, 16
CHUNK = 128
DHALF = 32
ROWS_PT = NPAD // NS
EDGES_PT_CONV = EPAD // NS
NCHUNK_CONV = EDGES_PT_CONV // CHUNK
SUPER = 2
SEDGE = SUPER * CHUNK
NSUP = EDGES_PT_CONV // SEDGE
NBUF = 3
IBUF = 4
IDXR = EPAD // CHUNK
ROWS_PT_IDX = EDGES_PT_CONV // CHUNK
EDGES_PT_P0 = EPAD // (NC * NS)
NCHUNK_P0 = EDGES_PT_P0 // CHUNK
SUP0 = 7
NSUP0 = NCHUNK_P0 // SUP0
ZROWS = 50
WCH = 128
RBLK = 2048
GRID_N = NPAD // RBLK
QPAD = NPAD // 4
QBLK = RBLK // 4

def _sc_mesh():
    return plsc.VectorSubcoreMesh(
        core_axis_name="c", subcore_axis_name="s",
        num_cores=NC, num_subcores=NS)


def _leaky(v):
    return jnp.where(v >= 0, v, 0.01 * v)


def _slot(n):
    return (n & -2048) | ((n & 511) << 2) | ((n >> 9) & 3)


def _phase0(src2, dst2):
    @functools.partial(
        pl.kernel,
        out_type=(
            jax.ShapeDtypeStruct((IDXR, CHUNK), jnp.int32),
            jax.ShapeDtypeStruct((2 * IDXR, CHUNK), jnp.int32),
            jax.ShapeDtypeStruct((NC * NPAD,), jnp.float32),
        ),
        mesh=_sc_mesh(),
        compiler_params=pltpu.CompilerParams(use_tc_tiling_on_sc=False),
        scratch_types=[
            pltpu.VMEM((SUP0, CHUNK), jnp.int32),
            pltpu.VMEM((SUP0, CHUNK), jnp.int32),
            pltpu.VMEM((2, SUP0, CHUNK), jnp.int32),
            pltpu.VMEM((2, SUP0, CHUNK), jnp.int32),
            pltpu.VMEM((2, SUP0, CHUNK), jnp.int32),
            pltpu.VMEM((2, SUP0, CHUNK), jnp.int32),
            pltpu.VMEM((CHUNK,), jnp.float32),
            pltpu.VMEM((ROWS_PT,), jnp.float32),
            pltpu.VMEM_SHARED((NPAD,), jnp.float32),
            pltpu.SemaphoreType.DMA,
            pltpu.SemaphoreType.DMA,
            pltpu.SemaphoreType.DMA,
        ],
    )
    def k(src_hbm, dst_hbm, dstm2, srcb2, deg_hbm,
          src_v, dst_v, soff_v, shi_v, srcm_v, dstm_v, ones_v, zbuf_v, deg_sh,
          lsem, stsem, scsem):
        c = lax.axis_index("c")
        s = lax.axis_index("s")
        lane = lax.iota(jnp.int32, 16)
        dummy = NNODE + (lane & 7)

        for j in range(CHUNK // 16):
            ones_v[pl.ds(j * 16, 16)] = jnp.full((16,), 1.0, jnp.float32)

        def zfill(i, _):
            zbuf_v[pl.ds(i * 16, 16)] = jnp.zeros((16,), jnp.float32)
            return 0
        lax.fori_loop(0, ROWS_PT // 16, zfill, 0)

        pltpu.sync_copy(zbuf_v, deg_sh.at[pl.ds(s * ROWS_PT, ROWS_PT)])
        plsc.subcore_barrier()

        base_row = (c * NS + s) * NCHUNK_P0

        def fire_stores(i, b):
            r0 = base_row + i * SUP0
            pltpu.async_copy(dstm_v.at[b], dstm2.at[pl.ds(r0, SUP0)], stsem)
            pltpu.async_copy(soff_v.at[b], srcb2.at[pl.ds(r0, SUP0)], stsem)
            pltpu.async_copy(shi_v.at[b],
                             srcb2.at[pl.ds(IDXR + r0, SUP0)], stsem)

        def drain_stores(i, b):
            r0 = base_row + i * SUP0
            pltpu.make_async_copy(
                dstm_v.at[b], dstm2.at[pl.ds(r0, SUP0)], stsem).wait()
            pltpu.make_async_copy(
                soff_v.at[b], srcb2.at[pl.ds(r0, SUP0)], stsem).wait()
            pltpu.make_async_copy(
                shi_v.at[b], srcb2.at[pl.ds(IDXR + r0, SUP0)], stsem).wait()

        def fire_deg(b):
            for kk in range(SUP0):
                pltpu.async_copy(ones_v, deg_sh.at[srcm_v.at[b, kk]], scsem,
                                 add=True)

        def drain_deg(b):
            for kk in range(SUP0):
                pltpu.make_async_copy(
                    ones_v, deg_sh.at[srcm_v.at[b, kk]], scsem).wait()

        def step(i, b):
            r0 = base_row + i * SUP0
            @pl.when(i >= 2)
            def _():
                drain_stores(i - 2, b)
                drain_deg(b)
            pltpu.async_copy(src_hbm.at[pl.ds(r0, SUP0)], src_v, lsem)
            pltpu.async_copy(dst_hbm.at[pl.ds(r0, SUP0)], dst_v, lsem)
            pltpu.make_async_copy(
                src_hbm.at[pl.ds(r0, SUP0)], src_v, lsem).wait()
            pltpu.make_async_copy(
                dst_hbm.at[pl.ds(r0, SUP0)], dst_v, lsem).wait()
            for kk in range(SUP0):
                for j in range(CHUNK // 16):
                    sl = pl.ds(j * 16, 16)
                    sv = src_v[kk, sl]
                    dv = dst_v[kk, sl]
                    m = sv == dv
                    ss = _slot(sv)
                    soff_v[b, kk, sl] = ss
                    shi_v[b, kk, sl] = ss + NPAD
                    srcm_v[b, kk, sl] = jnp.where(m, dummy, sv)
                    dstm_v[b, kk, sl] = _slot(jnp.where(m, dummy, dv))
            fire_stores(i, b)
            fire_deg(b)

        def outer(o, _):
            step(2 * o, 0)
            step(2 * o + 1, 1)
            return 0
        lax.fori_loop(0, NSUP0 // 2, outer, 0)

        drain_stores(NSUP0 - 2, 0)
        drain_deg(0)
        drain_stores(NSUP0 - 1, 1)
        drain_deg(1)

        plsc.subcore_barrier()
        pltpu.sync_copy(deg_sh.at[pl.ds(s * ROWS_PT, ROWS_PT)], zbuf_v)
        pltpu.sync_copy(zbuf_v,
                        deg_hbm.at[pl.ds(c * NPAD + s * ROWS_PT, ROWS_PT)])

    return k(src2, dst2)


def _conv_sc(y_flat, srcb2, dstm2):
    @functools.partial(
        pl.kernel,
        out_type=jax.ShapeDtypeStruct((NC * NPAD, DHALF), jnp.float32),
        mesh=_sc_mesh(),
        compiler_params=pltpu.CompilerParams(use_tc_tiling_on_sc=False),
        scratch_types=[
            pltpu.VMEM((IBUF, SUPER, CHUNK), jnp.int32),
            pltpu.VMEM((IBUF, SUPER, CHUNK), jnp.int32),
            pltpu.VMEM((NBUF, SEDGE, DHALF), jnp.float32),
            pltpu.VMEM((ZROWS, DHALF), jnp.float32),
            pltpu.VMEM_SHARED((NPAD, DHALF), jnp.float32),
            pltpu.SemaphoreType.DMA,
            pltpu.SemaphoreType.DMA,
            pltpu.SemaphoreType.DMA,
        ],
    )
    def k(y_hbm, src2, dst2, out_hbm,
          sidx_v, didx_v, rows_v, zbuf_v, acc_sh, isem, gsem, ssem):
        c = lax.axis_index("c")
        s = lax.axis_index("s")

        def zfill(i, _):
            zbuf_v[i, pl.ds(0, 16)] = jnp.zeros((16,), jnp.float32)
            zbuf_v[i, pl.ds(16, 16)] = jnp.zeros((16,), jnp.float32)
            return 0
        lax.fori_loop(0, ZROWS, zfill, 0)

        def zrow(t):
            return acc_sh.at[pl.ds(s * ROWS_PT + t * ZROWS, ZROWS)]

        def zfire(t, _):
            pltpu.async_copy(zbuf_v, zrow(t), ssem)
            return 0
        lax.fori_loop(0, ROWS_PT // ZROWS, zfire, 0)

        def zdrain(t, _):
            pltpu.make_async_copy(zbuf_v, zrow(t), ssem).wait()
            return 0
        lax.fori_loop(0, ROWS_PT // ZROWS, zdrain, 0)
        plsc.subcore_barrier()

        base_row = s * ROWS_PT_IDX

        def fire_idx(j, bi):
            r0 = base_row + j * SUPER
            pltpu.async_copy(src2.at[pl.ds(c * IDXR + r0, SUPER)],
                             sidx_v.at[bi], isem)
            pltpu.async_copy(dst2.at[pl.ds(r0, SUPER)], didx_v.at[bi], isem)

        def wait_idx(j, bi):
            r0 = base_row + j * SUPER
            pltpu.make_async_copy(src2.at[pl.ds(c * IDXR + r0, SUPER)],
                                  sidx_v.at[bi], isem).wait()
            pltpu.make_async_copy(dst2.at[pl.ds(r0, SUPER)], didx_v.at[bi],
                                  isem).wait()

        def fire_gathers(b, bi):
            for kk in range(SUPER):
                pltpu.async_copy(
                    y_hbm.at[sidx_v.at[bi, kk]],
                    rows_v.at[b, pl.ds(kk * CHUNK, CHUNK)], gsem)

        def drain_gathers(b, bi):
            pltpu.make_async_copy(
                y_hbm.at[pl.ds(0, SEDGE)], rows_v.at[b], gsem).wait()

        def fire_scatters(b, bi):
            for kk in range(SUPER):
                pltpu.async_copy(
                    rows_v.at[b, pl.ds(kk * CHUNK, CHUNK)],
                    acc_sh.at[didx_v.at[bi, kk]], ssem, add=True)

        def drain_scatters(b, bi):
            pltpu.make_async_copy(
                rows_v.at[b], acc_sh.at[pl.ds(0, SEDGE)], ssem).wait()

        def step(i, p):
            b = p % NBUF
            b2 = (p + NBUF - 2) % NBUF
            bi = p % IBUF
            bi2 = (p + IBUF - 2) % IBUF
            bi3 = (p + IBUF - 3) % IBUF
            bin1 = (p + 1) % IBUF
            @pl.when(jnp.logical_and(i >= 3, i <= NSUP + 2))
            def _():
                drain_scatters(b, bi3)
            @pl.when(jnp.logical_and(i >= 2, i < NSUP + 2))
            def _():
                drain_gathers(b2, bi2)
                fire_scatters(b2, bi2)
            @pl.when(i + 1 < NSUP)
            def _():
                fire_idx(i + 1, bin1)
            @pl.when(i < NSUP)
            def _():
                wait_idx(i, bi)
                fire_gathers(b, bi)

        fire_idx(0, 0)

        def outer(o, _):
            for p in range(NBUF * IBUF):
                step(NBUF * IBUF * o + p, p)
            return 0
        lax.fori_loop(0, (NSUP + 2 + NBUF * IBUF) // (NBUF * IBUF), outer, 0)
        plsc.subcore_barrier()

        NW = ROWS_PT // WCH
        def wslot(t):
            return rows_v.at[t % NBUF,
                             pl.ds(((t // NBUF) % SUPER) * CHUNK, CHUNK)]
        for t in range(NW):
            r0 = s * ROWS_PT + t * WCH
            ho = c * NPAD + r0
            if t >= NBUF * SUPER:
                po = c * NPAD + s * ROWS_PT + (t - NBUF * SUPER) * WCH
                pltpu.make_async_copy(
                    wslot(t - NBUF * SUPER),
                    out_hbm.at[pl.ds(po, WCH)], isem).wait()
            pltpu.sync_copy(acc_sh.at[pl.ds(r0, WCH)], wslot(t))
            pltpu.async_copy(wslot(t), out_hbm.at[pl.ds(ho, WCH)], isem)
        for t in range(max(0, NW - NBUF * SUPER), NW):
            ho = c * NPAD + s * ROWS_PT + t * WCH
            pltpu.make_async_copy(
                wslot(t), out_hbm.at[pl.ds(ho, WCH)], isem).wait()

    return k(y_flat, srcb2, dstm2)


def _mlp_tc(features, mlp_w, mlp_b2):
    def body(f_ref, w_ref, b_ref, o_ref):
        o_ref[...] = (
            jnp.dot(f_ref[...], w_ref[...], preferred_element_type=jnp.float32)
            + b_ref[...])
    return pl.pallas_call(
        body,
        grid=(20,),
        in_specs=[
            pl.BlockSpec((2000, 128), lambda i: (i, 0)),
            pl.BlockSpec((128, 128), lambda i: (0, 0)),
            pl.BlockSpec((1, 128), lambda i: (0, 0)),
        ],
        out_specs=pl.BlockSpec((2000, 128), lambda i: (i, 0)),
        out_shape=jax.ShapeDtypeStruct((40000, 128), jnp.float32),
    )(features, mlp_w, mlp_b2)


def _dinv_of(deg_ref):
    deg = deg_ref[0] + deg_ref[1] + 1.0
    return lax.rsqrt(deg)[:, None]


def _l2n(h):
    s = jnp.sum(h * h, axis=1, keepdims=True)
    return h * lax.rsqrt(jnp.maximum(s, 1e-24))


def _split_out(o_ref, y):
    q = y.shape[0] // 4
    for c in range(2):
        cols = y[:, c * DHALF:(c + 1) * DHALF]
        o_ref[c] = jnp.concatenate(
            [cols[j * q:(j + 1) * q, :] for j in range(4)], axis=1)


def _unpack(h):
    return jnp.concatenate(
        [h[:, j * DHALF:(j + 1) * DHALF] for j in range(4)], axis=0)


def _cat2(ref):
    return jnp.concatenate([_unpack(ref[0]), _unpack(ref[1])], axis=1)


def _cat2sum(a_ref, y_ref):
    return jnp.concatenate(
        [_unpack(a_ref[0] + y_ref[0]), _unpack(a_ref[1] + y_ref[1])], axis=1)


def _layer0_tc(xcat, deg_part, cw1):
    def body(x_ref, deg_ref, w_ref, o_ref):
        dinv = _dinv_of(deg_ref)
        xn = _l2n(x_ref[...])
        y = jnp.dot(xn, w_ref[...], preferred_element_type=jnp.float32) * dinv
        _split_out(o_ref, y)
    return pl.pallas_call(
        body,
        grid=(GRID_N,),
        in_specs=[
            pl.BlockSpec((RBLK, 128), lambda i: (i, 0)),
            pl.BlockSpec((2, RBLK), lambda i: (0, i)),
            pl.BlockSpec((128, 64), lambda i: (0, 0)),
        ],
        out_specs=pl.BlockSpec((2, QBLK, 128), lambda i: (0, i, 0)),
        out_shape=jax.ShapeDtypeStruct((2, QPAD, 128), jnp.float32),
    )(xcat, deg_part, cw1)


def _layer_tc(acc, y, deg_part, cb2, gw, gb2, cwn):
    def body(a_ref, y_ref, deg_ref, cb_ref, gw_ref, gb_ref, wn_ref, o_ref):
        dinv = _dinv_of(deg_ref)
        h = dinv * _cat2sum(a_ref, y_ref) + cb_ref[...]
        h = _leaky(_l2n(h))
        x = _leaky(
            jnp.dot(h, gw_ref[...], preferred_element_type=jnp.float32)
            + gb_ref[...])
        yn = jnp.dot(x, wn_ref[...], preferred_element_type=jnp.float32) * dinv
        _split_out(o_ref, yn)
    blk2 = pl.BlockSpec((2, QBLK, 128), lambda i: (0, i, 0))
    w64 = pl.BlockSpec((64, 64), lambda i: (0, 0))
    b64 = pl.BlockSpec((1, 64), lambda i: (0, 0))
    return pl.pallas_call(
        body,
        grid=(GRID_N,),
        in_specs=[blk2, blk2, pl.BlockSpec((2, RBLK), lambda i: (0, i)),
                  b64, w64, b64, w64],
        out_specs=blk2,
        out_shape=jax.ShapeDtypeStruct((2, QPAD, 128), jnp.float32),
    )(acc, y, deg_part, cb2, gw, gb2, cwn)


def _layer3_tc(acc, y, deg_part, cb2, gw, gb2, cw4, cw5):
    def body(a_ref, y_ref, deg_ref, cb_ref, gw_ref, gb_ref, w4_ref, w5_ref,
             o4_ref, o5_ref, ox_ref):
        dinv = _dinv_of(deg_ref)
        h = dinv * _cat2sum(a_ref, y_ref) + cb_ref[...]
        h = _leaky(_l2n(h))
        x = _leaky(
            jnp.dot(h, gw_ref[...], preferred_element_type=jnp.float32)
            + gb_ref[...])
        y4 = jnp.dot(x, w4_ref[...], preferred_element_type=jnp.float32) * dinv
        y5 = jnp.dot(x, w5_ref[...], preferred_element_type=jnp.float32) * dinv
        _split_out(o4_ref, y4)
        _split_out(o5_ref, y5)
        ox_ref[...] = x
    blk2 = pl.BlockSpec((2, QBLK, 128), lambda i: (0, i, 0))
    w64 = pl.BlockSpec((64, 64), lambda i: (0, 0))
    b64 = pl.BlockSpec((1, 64), lambda i: (0, 0))
    return pl.pallas_call(
        body,
        grid=(GRID_N,),
        in_specs=[blk2, blk2, pl.BlockSpec((2, RBLK), lambda i: (0, i)),
                  b64, w64, b64, w64, w64],
        out_specs=[blk2, blk2, pl.BlockSpec((RBLK, 64), lambda i: (i, 0))],
        out_shape=[
            jax.ShapeDtypeStruct((2, QPAD, 128), jnp.float32),
            jax.ShapeDtypeStruct((2, QPAD, 128), jnp.float32),
            jax.ShapeDtypeStruct((NPAD, 64), jnp.float32),
        ],
    )(acc, y, deg_part, cb2, gw, gb2, cw4, cw5)


def _head_tc(acc, y, x3, deg_part, cb, gw, gb, lw, lb):
    def body(a_ref, yh_ref, x_ref, deg_ref, cb_ref, gw_ref, gb_ref,
             lw_ref, lb_ref, o_ref):
        dinv = _dinv_of(deg_ref)
        h = dinv * _cat2sum(a_ref, yh_ref) + cb_ref[...]
        h = _leaky(_l2n(h))
        xh = _leaky(
            jnp.dot(x_ref[...], lw_ref[...], preferred_element_type=jnp.float32)
            + lb_ref[...])
        res = (jnp.dot(h, gw_ref[...], preferred_element_type=jnp.float32)
               + gb_ref[...] + xh)
        o_ref[...] = res.T
    blk2 = pl.BlockSpec((2, QBLK, 128), lambda i: (0, i, 0))
    blkx = pl.BlockSpec((RBLK, 64), lambda i: (i, 0))
    w64 = pl.BlockSpec((64, 64), lambda i: (0, 0))
    b64 = pl.BlockSpec((1, 64), lambda i: (0, 0))
    return pl.pallas_call(
        body,
        grid=(GRID_N,),
        in_specs=[blk2, blk2, blkx,
                  pl.BlockSpec((2, RBLK), lambda i: (0, i)),
                  b64, w64, b64, w64, b64],
        out_specs=pl.BlockSpec((64, RBLK), lambda i: (0, i)),
        out_shape=jax.ShapeDtypeStruct((64, NNODE), jnp.float32),
    )(acc, y, x3, deg_part, cb, gw, gb, lw, lb).T


def kernel(features, edge_index, preference, mlp_w, mlp_b,
           cw1, cb1, cw2, cb2, cw3, cb3, cw4, cb4, cw5, cb5,
           lw1, lb1, lw2, lb2, lw3, lb3, lw4, lb4, lw5, lb5,
           gw1, gb1, gw2, gb2, gw3, gb3, gw4, gb4, gw5, gb5):
    pad = NNODE + (jnp.arange(EPAD - E0, dtype=jnp.int32) % 8)
    src_p = jnp.concatenate([edge_index[0], pad]).reshape(IDXR, CHUNK)
    dst_p = jnp.concatenate([edge_index[1], pad]).reshape(IDXR, CHUNK)

    dstm2, srcb2, deg_flat = _phase0(src_p, dst_p)
    deg_part = deg_flat.reshape(2, NPAD)

    r2 = lambda b: b.reshape(1, -1)

    temp = _mlp_tc(features, mlp_w, r2(mlp_b))
    xcat = jnp.concatenate(
        [preference, temp, jnp.zeros((NPAD - NNODE, 128), jnp.float32)], axis=0)

    def sc_in(y):
        return y.reshape(NC * NPAD, DHALF)

    def tc_in(a):
        return a.reshape(2, QPAD, 128)

    y1 = _layer0_tc(xcat, deg_part, cw1)
    acc1 = _conv_sc(sc_in(y1), srcb2, dstm2)
    y2 = _layer_tc(tc_in(acc1), y1, deg_part, r2(cb1), gw1, r2(gb1), cw2)
    acc2 = _conv_sc(sc_in(y2), srcb2, dstm2)
    y3 = _layer_tc(tc_in(acc2), y2, deg_part, r2(cb2), gw2, r2(gb2), cw3)
    acc3 = _conv_sc(sc_in(y3), srcb2, dstm2)
    y4, y5, x3 = _layer3_tc(tc_in(acc3), y3, deg_part,
                            r2(cb3), gw3, r2(gb3), cw4, cw5)
    acc4 = _conv_sc(sc_in(y4), srcb2, dstm2)
    acc5 = _conv_sc(sc_in(y5), srcb2, dstm2)
    mu = _head_tc(tc_in(acc4), y4, x3, deg_part,
                  r2(cb4), gw4, r2(gb4), lw4, r2(lb4))
    lv = _head_tc(tc_in(acc5), y5, x3, deg_part,
                  r2(cb5), gw5, r2(gb5), lw5, r2(lb5))
    return (mu, lv)
